# initial kernel scaffold (unmeasured)
import jax
import jax.numpy as jnp
from jax import lax
from jax.experimental import pallas as pl
from jax.experimental.pallas import tpu as pltpu


def kernel(
    x,
):
    def body(*refs):
        pass

    out_shape = jax.ShapeDtypeStruct(..., jnp.float32)
    return pl.pallas_call(body, out_shape=out_shape)(...)


# baseline (device time: 12666 ns/iter reference)
import jax
import jax.numpy as jnp
from jax import lax
from jax.experimental import pallas as pl
from jax.experimental.pallas import tpu as pltpu

N_DEV = 4


def kernel(x):
    m_per, n = x.shape

    def body(x_ref, out_ref, comm_ref, send_sems, recv_sems):
        my_pos = lax.axis_index("i")
        left = (my_pos - 1) % N_DEV
        right = (my_pos + 1) % N_DEV

        barrier_sem = pltpu.get_barrier_semaphore()
        for nbr in [left, right]:
            pl.semaphore_signal(
                barrier_sem, inc=1,
                device_id=(nbr,), device_id_type=pl.DeviceIdType.MESH,
            )
        pl.semaphore_wait(barrier_sem, 2)

        partial = jnp.max(x_ref[:, :], axis=0, keepdims=True)
        comm_ref[0, :, :] = partial
        acc = partial

        for h in range(N_DEV - 1):
            send_slot = h % 2
            recv_slot = (h + 1) % 2
            rdma = pltpu.make_async_remote_copy(
                src_ref=comm_ref.at[send_slot],
                dst_ref=comm_ref.at[recv_slot],
                send_sem=send_sems.at[send_slot],
                recv_sem=recv_sems.at[recv_slot],
                device_id=(right,),
                device_id_type=pl.DeviceIdType.MESH,
            )
            rdma.start()
            rdma.wait()
            acc = jnp.maximum(acc, comm_ref[recv_slot, :, :])

        out_ref[:, :] = acc

    return pl.pallas_call(
        body,
        out_shape=jax.ShapeDtypeStruct((1, n), jnp.float32),
        in_specs=[pl.BlockSpec(memory_space=pltpu.VMEM)],
        out_specs=pl.BlockSpec(memory_space=pltpu.VMEM),
        scratch_shapes=[
            pltpu.VMEM((2, 1, n), jnp.float32),
            pltpu.SemaphoreType.DMA((2,)),
            pltpu.SemaphoreType.DMA((2,)),
        ],
        compiler_params=pltpu.CompilerParams(collective_id=0),
    )(x)


# device time: 10582 ns/iter; 1.1969x vs baseline; 1.1969x over previous
import jax
import jax.numpy as jnp
from jax import lax
from jax.experimental import pallas as pl
from jax.experimental.pallas import tpu as pltpu

N_DEV = 4


def kernel(x):
    m_per, n = x.shape

    def body(x_ref, out_ref, acc_ref, comm_ref, send_sems, recv_sems):
        my_pos = lax.axis_index("i")
        left = (my_pos - 1) % N_DEV
        right = (my_pos + 1) % N_DEV

        barrier_sem = pltpu.get_barrier_semaphore()
        for nbr in [left, right]:
            pl.semaphore_signal(
                barrier_sem, inc=1,
                device_id=(nbr,), device_id_type=pl.DeviceIdType.MESH,
            )
        pl.semaphore_wait(barrier_sem, 2)

        acc_ref[:, :] = jnp.max(x_ref[:, :], axis=0, keepdims=True)

        partners = [my_pos ^ 1, 3 - my_pos]
        for s, partner in enumerate(partners):
            rdma = pltpu.make_async_remote_copy(
                src_ref=acc_ref,
                dst_ref=comm_ref.at[s],
                send_sem=send_sems.at[s],
                recv_sem=recv_sems.at[s],
                device_id=(partner,),
                device_id_type=pl.DeviceIdType.MESH,
            )
            rdma.start()
            rdma.wait()
            acc_ref[:, :] = jnp.maximum(acc_ref[:, :], comm_ref[s, :, :])

        out_ref[:, :] = acc_ref[:, :]

    return pl.pallas_call(
        body,
        out_shape=jax.ShapeDtypeStruct((1, n), jnp.float32),
        in_specs=[pl.BlockSpec(memory_space=pltpu.VMEM)],
        out_specs=pl.BlockSpec(memory_space=pltpu.VMEM),
        scratch_shapes=[
            pltpu.VMEM((1, n), jnp.float32),
            pltpu.VMEM((2, 1, n), jnp.float32),
            pltpu.SemaphoreType.DMA((2,)),
            pltpu.SemaphoreType.DMA((2,)),
        ],
        compiler_params=pltpu.CompilerParams(collective_id=0),
    )(x)


# device time: 9668 ns/iter; 1.3101x vs baseline; 1.0945x over previous
import jax
import jax.numpy as jnp
from jax import lax
from jax.experimental import pallas as pl
from jax.experimental.pallas import tpu as pltpu

N_DEV = 4
CHUNK = 512


def kernel(x):
    m_per, n = x.shape
    n_chunks = m_per // CHUNK

    def body(x_hbm, out_ref, acc_ref, vbuf, comm_ref,
             copy_sems, send_sems, recv_sems):
        my_pos = lax.axis_index("i")
        peers = [(my_pos + k) % N_DEV for k in (1, 2, 3)]

        barrier_sem = pltpu.get_barrier_semaphore()
        for p in peers:
            pl.semaphore_signal(
                barrier_sem, inc=1,
                device_id=(p,), device_id_type=pl.DeviceIdType.MESH,
            )

        def copy_in(i):
            return pltpu.make_async_copy(
                x_hbm.at[pl.ds(i * CHUNK, CHUNK), :],
                vbuf.at[i % 2],
                copy_sems.at[i % 2],
            )

        copy_in(0).start()
        copy_in(1).start()
        acc = None
        for i in range(n_chunks):
            copy_in(i).wait()
            if i + 2 < n_chunks:
                copy_in(i + 2).start()
            part = jnp.max(vbuf[i % 2], axis=0, keepdims=True)
            acc = part if acc is None else jnp.maximum(acc, part)
        acc_ref[:, :] = acc

        pl.semaphore_wait(barrier_sem, 3)

        rdmas = []
        for k in (1, 2, 3):
            r = pltpu.make_async_remote_copy(
                src_ref=acc_ref,
                dst_ref=comm_ref.at[3 - k],
                send_sem=send_sems.at[k - 1],
                recv_sem=recv_sems.at[3 - k],
                device_id=(peers[k - 1],),
                device_id_type=pl.DeviceIdType.MESH,
            )
            r.start()
            rdmas.append(r)
        for r in rdmas:
            r.wait()

        out_ref[:, :] = jnp.maximum(
            jnp.maximum(acc, comm_ref[0, :, :]),
            jnp.maximum(comm_ref[1, :, :], comm_ref[2, :, :]),
        )

    return pl.pallas_call(
        body,
        out_shape=jax.ShapeDtypeStruct((1, n), jnp.float32),
        in_specs=[pl.BlockSpec(memory_space=pltpu.MemorySpace.HBM)],
        out_specs=pl.BlockSpec(memory_space=pltpu.VMEM),
        scratch_shapes=[
            pltpu.VMEM((1, n), jnp.float32),
            pltpu.VMEM((2, CHUNK, n), jnp.float32),
            pltpu.VMEM((3, 1, n), jnp.float32),
            pltpu.SemaphoreType.DMA((2,)),
            pltpu.SemaphoreType.DMA((3,)),
            pltpu.SemaphoreType.DMA((3,)),
        ],
        compiler_params=pltpu.CompilerParams(collective_id=0),
    )(x)


# device time: 9651 ns/iter; 1.3124x vs baseline; 1.0018x over previous
import jax
import jax.numpy as jnp
from jax import lax
from jax.experimental import pallas as pl
from jax.experimental.pallas import tpu as pltpu

N_DEV = 4
CHUNK = 512


def kernel(x):
    m_per, n = x.shape
    n_chunks = m_per // CHUNK

    def body(x_hbm, out_ref, acc_ref, vbuf, comm_ref,
             copy_sems, send_sems, recv_sems):
        my_pos = lax.axis_index("i")
        peers = [(my_pos + k) % N_DEV for k in (1, 2, 3)]

        barrier_sem = pltpu.get_barrier_semaphore()
        for p in peers:
            pl.semaphore_signal(
                barrier_sem, inc=1,
                device_id=(p,), device_id_type=pl.DeviceIdType.MESH,
            )

        def copy_in(i):
            return pltpu.make_async_copy(
                x_hbm.at[pl.ds(i * CHUNK, CHUNK), :],
                vbuf.at[i % 2],
                copy_sems.at[i % 2],
            )

        copy_in(0).start()
        copy_in(1).start()
        acc = None
        for i in range(n_chunks):
            copy_in(i).wait()
            part = jnp.max(vbuf[i % 2], axis=0, keepdims=True)
            acc = part if acc is None else jnp.maximum(acc, part)
            if i + 2 < n_chunks:
                copy_in(i + 2).start()
        acc_ref[:, :] = acc

        pl.semaphore_wait(barrier_sem, 3)

        rdmas = []
        for k in (1, 2, 3):
            r = pltpu.make_async_remote_copy(
                src_ref=acc_ref,
                dst_ref=comm_ref.at[3 - k],
                send_sem=send_sems.at[k - 1],
                recv_sem=recv_sems.at[3 - k],
                device_id=(peers[k - 1],),
                device_id_type=pl.DeviceIdType.MESH,
            )
            r.start()
            rdmas.append(r)
        for r in rdmas:
            r.wait()

        out_ref[:, :] = jnp.maximum(
            jnp.maximum(acc, comm_ref[0, :, :]),
            jnp.maximum(comm_ref[1, :, :], comm_ref[2, :, :]),
        )

    return pl.pallas_call(
        body,
        out_shape=jax.ShapeDtypeStruct((1, n), jnp.float32),
        in_specs=[pl.BlockSpec(memory_space=pltpu.MemorySpace.HBM)],
        out_specs=pl.BlockSpec(memory_space=pltpu.VMEM),
        scratch_shapes=[
            pltpu.VMEM((1, n), jnp.float32),
            pltpu.VMEM((2, CHUNK, n), jnp.float32),
            pltpu.VMEM((3, 1, n), jnp.float32),
            pltpu.SemaphoreType.DMA((2,)),
            pltpu.SemaphoreType.DMA((3,)),
            pltpu.SemaphoreType.DMA((3,)),
        ],
        compiler_params=pltpu.CompilerParams(collective_id=0),
    )(x)
